# baseline (speedup 1.0000x reference)
import functools

import jax
import jax.numpy as jnp
from jax import lax
from jax.experimental import pallas as pl
from jax.experimental.pallas import tpu as pltpu
from jax.experimental.pallas import tpu_sc as plsc

TEMP = 0.07
MOM = 0.5
EPS = 1e-07

NC = 2
NS = 16
NW = NC * NS
LANES = 16

_GDN = lax.GatherDimensionNumbers(
    offset_dims=(), collapsed_slice_dims=(0,), start_index_map=(0,))


def _shuffle(x, idx):
    return lax.gather(x, idx[:, None], _GDN, slice_sizes=(1,),
                      mode=lax.GatherScatterMode.PROMISE_IN_BOUNDS)


def _proj_body(xs_ref, ws_ref, bs_ref, xt_ref, wt_ref, bt_ref, os_ref, ot_ref):
    for x_ref, w_ref, b_ref, o_ref in ((xs_ref, ws_ref, bs_ref, os_ref),
                                       (xt_ref, wt_ref, bt_ref, ot_ref)):
        y = lax.dot_general(x_ref[...], w_ref[...], (((1,), (1,)), ((), ())),
                            preferred_element_type=jnp.float32,
                            precision=lax.Precision.HIGHEST)
        y = y + b_ref[...]
        n = jnp.sqrt(jnp.sum(y * y, axis=1, keepdims=True))
        o_ref[...] = y / jnp.maximum(n, 1e-12)


def _proj(fs_s, w_s, b_s, fs_t, w_t, b_t):
    B = fs_s.shape[0]
    F = w_s.shape[0]
    return pl.pallas_call(
        _proj_body,
        out_shape=[jax.ShapeDtypeStruct((B, F), jnp.float32),
                   jax.ShapeDtypeStruct((B, F), jnp.float32)],
    )(fs_s, w_s, b_s.reshape(1, F), fs_t, w_t, b_t.reshape(1, F))


def _pos_gather(idx, m1, m2):
    B = idx.shape[0]
    F = m1.shape[1]
    bpw = B // NW
    mesh = plsc.VectorSubcoreMesh(core_axis_name="c", subcore_axis_name="s")

    @functools.partial(
        pl.kernel, mesh=mesh,
        compiler_params=pltpu.CompilerParams(use_tc_tiling_on_sc=False, needs_layout_passes=False),
        out_type=[jax.ShapeDtypeStruct((B, F), jnp.float32),
                  jax.ShapeDtypeStruct((B, F), jnp.float32)],
        scratch_types=[pltpu.VMEM((bpw,), jnp.int32),
                       pltpu.VMEM((bpw, F), jnp.float32),
                       pltpu.SemaphoreType.DMA],
    )
    def k(idx_hbm, m1_hbm, m2_hbm, o1_hbm, o2_hbm, idx_v, rows_v, sem):
        wid = lax.axis_index("s") * NC + lax.axis_index("c")
        base = wid * bpw
        pltpu.sync_copy(idx_hbm.at[pl.ds(base, bpw)], idx_v)
        pltpu.async_copy(m1_hbm.at[idx_v], rows_v, sem).wait()
        pltpu.sync_copy(rows_v, o1_hbm.at[pl.ds(base, bpw)])
        pltpu.async_copy(m2_hbm.at[idx_v], rows_v, sem).wait()
        pltpu.sync_copy(rows_v, o2_hbm.at[pl.ds(base, bpw)])

    return k(idx, m1, m2)


def _prep_body(p1_ref, p2_ref, fs_ref, ft_ref, et_ref, es_ref):
    u_s = p1_ref[...] * MOM + fs_ref[...] * (1.0 - MOM)
    u_s = u_s / jnp.sqrt(jnp.sum(u_s * u_s, axis=1, keepdims=True))
    u_t = p2_ref[...] * MOM + ft_ref[...] * (1.0 - MOM)
    u_t = u_t / jnp.sqrt(jnp.sum(u_t * u_t, axis=1, keepdims=True))
    dn = (((1,), (1,)), ((), ()))
    et_ref[...] = lax.dot_general(ft_ref[...], u_s, dn,
                                  preferred_element_type=jnp.float32,
                                  precision=lax.Precision.HIGHEST)
    es_ref[...] = lax.dot_general(fs_ref[...], u_t, dn,
                                  preferred_element_type=jnp.float32,
                                  precision=lax.Precision.HIGHEST)


def _prep(pos1, pos2, f_s, f_t):
    B = pos1.shape[0]
    return pl.pallas_call(
        _prep_body,
        out_shape=[jax.ShapeDtypeStruct((B, B), jnp.float32),
                   jax.ShapeDtypeStruct((B, B), jnp.float32)],
    )(pos1, pos2, f_s, f_t)


def _sc_main(cidx_flat, idx, wtab, m1, m2, f_t, f_s, e_t, e_s):
    B, K = idx.shape[0], cidx_flat.shape[0] // idx.shape[0]
    F = m1.shape[1]
    bpw = B // NW
    CH = 64
    nch = K // CH
    mesh = plsc.VectorSubcoreMesh(core_axis_name="c", subcore_axis_name="s")

    @functools.partial(
        pl.kernel, mesh=mesh,
        compiler_params=pltpu.CompilerParams(use_tc_tiling_on_sc=False, needs_layout_passes=False),
        out_type=[jax.ShapeDtypeStruct((B * K,), jnp.float32),
                  jax.ShapeDtypeStruct((B * K,), jnp.float32),
                  jax.ShapeDtypeStruct((B,), jnp.float32),
                  jax.ShapeDtypeStruct((B,), jnp.float32)],
        scratch_types=[pltpu.VMEM((bpw, B), jnp.float32),
                       pltpu.VMEM((bpw, B), jnp.float32),
                       pltpu.VMEM((bpw, F), jnp.float32),
                       pltpu.VMEM((bpw, F), jnp.float32),
                       pltpu.VMEM((bpw,), jnp.int32),
                       pltpu.VMEM((bpw,), jnp.int32),
                       pltpu.VMEM((bpw * K,), jnp.int32),
                       pltpu.VMEM((CH,), jnp.int32),
                       pltpu.VMEM((CH,), jnp.int32),
                       pltpu.VMEM((CH, F), jnp.float32),
                       pltpu.VMEM((CH, F), jnp.float32),
                       pltpu.VMEM((CH,), jnp.float32),
                       pltpu.VMEM((CH,), jnp.float32),
                       pltpu.VMEM((bpw,), jnp.float32),
                       pltpu.VMEM((bpw,), jnp.float32),
                       pltpu.SemaphoreType.DMA,
                       pltpu.SemaphoreType.DMA,
                       pltpu.SemaphoreType.DMA,
                       pltpu.SemaphoreType.DMA,
                       pltpu.SemaphoreType.DMA,
                       pltpu.SemaphoreType.DMA,
                       pltpu.SemaphoreType.DMA],
    )
    def k(cidx_hbm, idx_hbm, wtab_hbm, m1_hbm, m2_hbm, ft_hbm, fs_hbm,
          et_hbm, es_hbm, dt_hbm, ds_hbm, ppt_hbm, pps_hbm,
          et_v, es_v, ft_v, fs_v, own_v, wown_v, ci_all, wc0, wc1,
          rows0, rows1, dots0, dots1, ppt_v, pps_v,
          sem, semr0, semr1, semw0, semw1, semd0, semd1):
        wid = lax.axis_index("s") * NC + lax.axis_index("c")
        s0 = wid * bpw
        pltpu.sync_copy(ft_hbm.at[pl.ds(s0, bpw)], ft_v)
        pltpu.sync_copy(fs_hbm.at[pl.ds(s0, bpw)], fs_v)
        pltpu.sync_copy(et_hbm.at[pl.ds(s0, bpw)], et_v)
        pltpu.sync_copy(es_hbm.at[pl.ds(s0, bpw)], es_v)
        pltpu.sync_copy(idx_hbm.at[pl.ds(s0, bpw)], own_v)
        pltpu.sync_copy(cidx_hbm.at[pl.ds(s0 * K, bpw * K)], ci_all)
        pltpu.async_copy(wtab_hbm.at[own_v], wown_v, sem).wait()

        lane = jnp.arange(LANES, dtype=jnp.int32)
        nchk = bpw * nch

        def run_bank(mem_hbm, o_hbm, f_v, e_v):
            def locs(c):
                i = c // nch
                lbase = i * K + (c - i * nch) * CH
                return i, lbase

            def issue(c, rows_b, wc_b, semr, semw):
                _, lbase = locs(c)
                idxref = ci_all.at[pl.ds(lbase, CH)]
                pltpu.async_copy(mem_hbm.at[idxref], rows_b, semr)

            def wait_in(c, rows_b, wc_b, semr, semw):
                _, lbase = locs(c)
                idxref = ci_all.at[pl.ds(lbase, CH)]
                pltpu.make_async_copy(mem_hbm.at[idxref], rows_b, semr).wait()

            def compute(c, rows_b, wc_b, dots_b):
                i, _ = locs(c)
                fsegs = [f_v[i, pl.ds(ss * LANES, LANES)]
                         for ss in range(F // LANES)]
                isplat = jnp.full((LANES,), i, jnp.int32)

                def per_group(rg, _):
                    dots = rows_b[rg, pl.ds(0, LANES)] * fsegs[0]
                    wvec = wc_b[pl.ds(rg * LANES, LANES)]
                    hit = wvec >= 0
                    corr = plsc.load_gather(e_v, [isplat, jnp.maximum(wvec, 0)])
                    dots_b[pl.ds(rg * LANES, LANES)] = jnp.where(hit, corr, dots)
                    return 0

                lax.fori_loop(0, CH // LANES, per_group, 0)

            def wb(c, dots_b, semd):
                _, lbase = locs(c)
                pltpu.async_copy(dots_b, o_hbm.at[pl.ds(s0 * K + lbase, CH)], semd)

            def wb_wait(c, dots_b, semd):
                _, lbase = locs(c)
                pltpu.make_async_copy(
                    dots_b, o_hbm.at[pl.ds(s0 * K + lbase, CH)], semd).wait()

            issue(0, rows0, wc0, semr0, semw0)

            def body(cc, _):
                c0 = 2 * cc
                issue(c0 + 1, rows1, wc1, semr1, semw1)
                wait_in(c0, rows0, wc0, semr0, semw0)
                pl.when(cc > 0)(lambda: wb_wait(c0 - 2, dots0, semd0))
                compute(c0, rows0, wc0, dots0)
                wb(c0, dots0, semd0)
                pl.when(cc < nchk // 2 - 1)(
                    lambda: issue(c0 + 2, rows0, wc0, semr0, semw0))
                wait_in(c0 + 1, rows1, wc1, semr1, semw1)
                pl.when(cc > 0)(lambda: wb_wait(c0 - 1, dots1, semd1))
                compute(c0 + 1, rows1, wc1, dots1)
                wb(c0 + 1, dots1, semd1)
                return 0

            lax.fori_loop(0, nchk // 2, body, 0)
            wb_wait(nchk - 2, dots0, semd0)
            wb_wait(nchk - 1, dots1, semd1)

        run_bank(m1_hbm, dt_hbm, ft_v, et_v)
        run_bank(m2_hbm, ds_hbm, fs_v, es_v)

        for grp in range(bpw // LANES):
            ivec = lane + grp * LANES
            wvec = wown_v[pl.ds(grp * LANES, LANES)]
            ppt_v[pl.ds(grp * LANES, LANES)] = plsc.load_gather(et_v, [ivec, wvec])
            pps_v[pl.ds(grp * LANES, LANES)] = plsc.load_gather(es_v, [ivec, wvec])
        pltpu.sync_copy(ppt_v, ppt_hbm.at[pl.ds(s0, bpw)])
        pltpu.sync_copy(pps_v, pps_hbm.at[pl.ds(s0, bpw)])

    return k(cidx_flat, idx, wtab, m1, m2, f_t, f_s, e_t, e_s)


def _loss_body(dt_ref, ds_ref, ppt_ref, pps_ref, o_ref, *, n_data, n_tot, b):
    residual = float(n_tot // b - 1) / n_data
    out = 0.0
    for pp_ref, dn_ref in ((pps_ref, ds_ref), (ppt_ref, dt_ref)):
        a0 = jnp.exp(pp_ref[...] / TEMP)
        an = jnp.exp(dn_ref[...] / TEMP)
        z = (jnp.sum(a0) + jnp.sum(an)) / n_tot * n_data
        o0 = a0 / z
        on = an / z
        ld1 = jnp.sum(jnp.log(o0 / (o0 + residual + EPS)))
        ld0 = jnp.sum(jnp.log(residual / (on + residual + EPS)))
        out = out - (ld1 + ld0) / b
    o_ref[...] = jnp.full((1, 1), 1.0) * out


def _loss(dots_t, dots_s, pp_t, pp_s, n_data):
    B = pp_t.shape[0] * pp_t.shape[1]
    n_tot = B + dots_t.shape[0] * dots_t.shape[1]
    body = functools.partial(_loss_body, n_data=n_data, n_tot=n_tot, b=B)
    return pl.pallas_call(
        body,
        out_shape=jax.ShapeDtypeStruct((1, 1), jnp.float32),
    )(dots_t, dots_s, pp_t, pp_s)


def kernel(fs_s, fs_t, idx, contrast_idx, W_s, b_s, W_t, b_t, memory_v1, memory_v2):
    B, K = contrast_idx.shape
    n_data = memory_v1.shape[0]
    idx32 = idx.astype(jnp.int32)
    cidx_flat = contrast_idx.reshape(-1).astype(jnp.int32)
    wtab = jnp.full((n_data,), -1, jnp.int32).at[idx32].set(
        jnp.arange(B, dtype=jnp.int32))

    f_s, f_t = _proj(fs_s, W_s, b_s, fs_t, W_t, b_t)
    pos1, pos2 = _pos_gather(idx32, memory_v1, memory_v2)
    e_t, e_s = _prep(pos1, pos2, f_s, f_t)
    dt_flat, ds_flat, pp_t, pp_s = _sc_main(
        cidx_flat, idx32, wtab, memory_v1, memory_v2, f_t, f_s, e_t, e_s)
    out = _loss(dt_flat.reshape(B * 2, K // 2), ds_flat.reshape(B * 2, K // 2),
                pp_t.reshape(8, B // 8), pp_s.reshape(8, B // 8), n_data)
    return out.reshape(())


# speedup vs baseline: 1.2791x; 1.2791x over previous
import functools

import jax
import jax.numpy as jnp
from jax import lax
from jax.experimental import pallas as pl
from jax.experimental.pallas import tpu as pltpu
from jax.experimental.pallas import tpu_sc as plsc

TEMP = 0.07
MOM = 0.5
EPS = 1e-07

NC = 2
NS = 16
NW = NC * NS
LANES = 16

_GDN = lax.GatherDimensionNumbers(
    offset_dims=(), collapsed_slice_dims=(0,), start_index_map=(0,))


def _shuffle(x, idx):
    return lax.gather(x, idx[:, None], _GDN, slice_sizes=(1,),
                      mode=lax.GatherScatterMode.PROMISE_IN_BOUNDS)


def _proj_body(xs_ref, ws_ref, bs_ref, xt_ref, wt_ref, bt_ref, os_ref, ot_ref):
    for x_ref, w_ref, b_ref, o_ref in ((xs_ref, ws_ref, bs_ref, os_ref),
                                       (xt_ref, wt_ref, bt_ref, ot_ref)):
        y = lax.dot_general(x_ref[...], w_ref[...], (((1,), (1,)), ((), ())),
                            preferred_element_type=jnp.float32,
                            precision=lax.Precision.HIGHEST)
        y = y + b_ref[...]
        n = jnp.sqrt(jnp.sum(y * y, axis=1, keepdims=True))
        o_ref[...] = y / jnp.maximum(n, 1e-12)


def _proj(fs_s, w_s, b_s, fs_t, w_t, b_t):
    B = fs_s.shape[0]
    F = w_s.shape[0]
    return pl.pallas_call(
        _proj_body,
        out_shape=[jax.ShapeDtypeStruct((B, F), jnp.float32),
                   jax.ShapeDtypeStruct((B, F), jnp.float32)],
    )(fs_s, w_s, b_s.reshape(1, F), fs_t, w_t, b_t.reshape(1, F))


def _pos_gather(idx, m1, m2):
    B = idx.shape[0]
    F = m1.shape[1]
    bpw = B // NW
    mesh = plsc.VectorSubcoreMesh(core_axis_name="c", subcore_axis_name="s")

    @functools.partial(
        pl.kernel, mesh=mesh,
        compiler_params=pltpu.CompilerParams(use_tc_tiling_on_sc=False, needs_layout_passes=False),
        out_type=[jax.ShapeDtypeStruct((B, F), jnp.float32),
                  jax.ShapeDtypeStruct((B, F), jnp.float32)],
        scratch_types=[pltpu.VMEM((bpw,), jnp.int32),
                       pltpu.VMEM((bpw, F), jnp.float32),
                       pltpu.SemaphoreType.DMA],
    )
    def k(idx_hbm, m1_hbm, m2_hbm, o1_hbm, o2_hbm, idx_v, rows_v, sem):
        wid = lax.axis_index("s") * NC + lax.axis_index("c")
        base = wid * bpw
        pltpu.sync_copy(idx_hbm.at[pl.ds(base, bpw)], idx_v)
        pltpu.async_copy(m1_hbm.at[idx_v], rows_v, sem).wait()
        pltpu.sync_copy(rows_v, o1_hbm.at[pl.ds(base, bpw)])
        pltpu.async_copy(m2_hbm.at[idx_v], rows_v, sem).wait()
        pltpu.sync_copy(rows_v, o2_hbm.at[pl.ds(base, bpw)])

    return k(idx, m1, m2)


def _prep_body(p1_ref, p2_ref, fs_ref, ft_ref, et_ref, es_ref):
    u_s = p1_ref[...] * MOM + fs_ref[...] * (1.0 - MOM)
    u_s = u_s / jnp.sqrt(jnp.sum(u_s * u_s, axis=1, keepdims=True))
    u_t = p2_ref[...] * MOM + ft_ref[...] * (1.0 - MOM)
    u_t = u_t / jnp.sqrt(jnp.sum(u_t * u_t, axis=1, keepdims=True))
    dn = (((1,), (1,)), ((), ()))
    et_ref[...] = lax.dot_general(ft_ref[...], u_s, dn,
                                  preferred_element_type=jnp.float32,
                                  precision=lax.Precision.HIGHEST)
    es_ref[...] = lax.dot_general(fs_ref[...], u_t, dn,
                                  preferred_element_type=jnp.float32,
                                  precision=lax.Precision.HIGHEST)


def _prep(pos1, pos2, f_s, f_t):
    B = pos1.shape[0]
    return pl.pallas_call(
        _prep_body,
        out_shape=[jax.ShapeDtypeStruct((B, B), jnp.float32),
                   jax.ShapeDtypeStruct((B, B), jnp.float32)],
    )(pos1, pos2, f_s, f_t)


def _sc_main(cidx_flat, idx, wtab, m1, m2, f_t, f_s, e_t, e_s):
    B, K = idx.shape[0], cidx_flat.shape[0] // idx.shape[0]
    F = m1.shape[1]
    bpw = B // NW
    CH = 128
    nch = K // CH
    nchk = bpw * nch
    NB = 4
    mesh = plsc.VectorSubcoreMesh(core_axis_name="c", subcore_axis_name="s")

    @functools.partial(
        pl.kernel, mesh=mesh,
        compiler_params=pltpu.CompilerParams(use_tc_tiling_on_sc=False, needs_layout_passes=False),
        out_type=[jax.ShapeDtypeStruct((B * K,), jnp.float32),
                  jax.ShapeDtypeStruct((B * K,), jnp.float32),
                  jax.ShapeDtypeStruct((B,), jnp.float32),
                  jax.ShapeDtypeStruct((B,), jnp.float32)],
        scratch_types=([pltpu.VMEM((bpw, F), jnp.float32),
                        pltpu.VMEM((bpw, F), jnp.float32),
                        pltpu.VMEM((bpw,), jnp.int32),
                        pltpu.VMEM((bpw,), jnp.int32),
                        pltpu.VMEM((bpw * K,), jnp.int32),
                        pltpu.VMEM((2, B), jnp.float32),
                        pltpu.VMEM((bpw,), jnp.float32),
                        pltpu.VMEM((bpw,), jnp.float32)]
                       + [pltpu.VMEM((CH,), jnp.int32)] * NB
                       + [pltpu.VMEM((CH, F), jnp.float32)] * NB
                       + [pltpu.VMEM((CH,), jnp.float32)] * NB
                       + [pltpu.SemaphoreType.DMA] * (1 + 3 * NB + 2)),
    )
    def k(cidx_hbm, idx_hbm, wtab_hbm, m1_hbm, m2_hbm, ft_hbm, fs_hbm,
          et_hbm, es_hbm, dt_hbm, ds_hbm, ppt_hbm, pps_hbm,
          ft_v, fs_v, own_v, wown_v, ci_all, ering, ppt_v, pps_v,
          wc0, wc1, wc2, wc3, rows0, rows1, rows2, rows3,
          dots0, dots1, dots2, dots3,
          sem, semr0, semr1, semr2, semr3, semw0, semw1, semw2, semw3,
          semd0, semd1, semd2, semd3, seme0, seme1):
        wc_bufs = (wc0, wc1, wc2, wc3)
        rows_bufs = (rows0, rows1, rows2, rows3)
        dots_bufs = (dots0, dots1, dots2, dots3)
        semr_t = (semr0, semr1, semr2, semr3)
        semw_t = (semw0, semw1, semw2, semw3)
        semd_t = (semd0, semd1, semd2, semd3)
        seme_t = (seme0, seme1)

        wid = lax.axis_index("s") * NC + lax.axis_index("c")
        s0 = wid * bpw
        pltpu.sync_copy(ft_hbm.at[pl.ds(s0, bpw)], ft_v)
        pltpu.sync_copy(fs_hbm.at[pl.ds(s0, bpw)], fs_v)
        pltpu.sync_copy(idx_hbm.at[pl.ds(s0, bpw)], own_v)
        pltpu.sync_copy(cidx_hbm.at[pl.ds(s0 * K, bpw * K)], ci_all)
        pltpu.async_copy(wtab_hbm.at[own_v], wown_v, sem).wait()

        lane = jnp.arange(LANES, dtype=jnp.int32)

        def run_bank(mem_hbm, o_hbm, f_v, e_hbm, pp_v):
            def locs(c):
                i = c // nch
                lbase = i * K + (c - i * nch) * CH
                return i, lbase

            def issue(c, j):
                _, lbase = locs(c)
                idxref = ci_all.at[pl.ds(lbase, CH)]
                pltpu.async_copy(wtab_hbm.at[idxref], wc_bufs[j], semw_t[j])
                pltpu.async_copy(mem_hbm.at[idxref], rows_bufs[j], semr_t[j])

            def wait_in(c, j):
                _, lbase = locs(c)
                idxref = ci_all.at[pl.ds(lbase, CH)]
                pltpu.make_async_copy(wtab_hbm.at[idxref], wc_bufs[j], semw_t[j]).wait()
                pltpu.make_async_copy(mem_hbm.at[idxref], rows_bufs[j], semr_t[j]).wait()

            def wb(c, j):
                _, lbase = locs(c)
                pltpu.async_copy(dots_bufs[j],
                                 o_hbm.at[pl.ds(s0 * K + lbase, CH)], semd_t[j])

            def wb_wait(c, j):
                _, lbase = locs(c)
                pltpu.make_async_copy(
                    dots_bufs[j], o_hbm.at[pl.ds(s0 * K + lbase, CH)],
                    semd_t[j]).wait()

            def issue_e(i, sl):
                pltpu.async_copy(e_hbm.at[s0 + i], ering.at[sl], seme_t[sl])

            def wait_e(i, sl):
                pltpu.make_async_copy(
                    e_hbm.at[s0 + i], ering.at[sl], seme_t[sl]).wait()

            def compute(c, j, sl):
                i, _ = locs(c)
                rows_b, wc_b, dots_b = rows_bufs[j], wc_bufs[j], dots_bufs[j]
                fsegs = [f_v[i, pl.ds(ss * LANES, LANES)]
                         for ss in range(F // LANES)]
                esplat = jnp.full((LANES,), sl, jnp.int32)

                def per_group(rg, _):
                    dots = jnp.zeros((LANES,), jnp.float32)
                    for rr in range(LANES):
                        r = rg * LANES + rr
                        acc = rows_b[r, pl.ds(0, LANES)] * fsegs[0]
                        for ss in range(1, F // LANES):
                            acc = acc + (rows_b[r, pl.ds(ss * LANES, LANES)]
                                         * fsegs[ss])
                        for step in (1, 2, 4, 8):
                            acc = acc + _shuffle(acc, lane ^ step)
                        dots = jnp.where(lane == rr, acc, dots)
                    wvec = wc_b[pl.ds(rg * LANES, LANES)]
                    hit = wvec >= 0
                    corr = plsc.load_gather(ering, [esplat, jnp.maximum(wvec, 0)])
                    dots_b[pl.ds(rg * LANES, LANES)] = jnp.where(hit, corr, dots)
                    return 0

                lax.fori_loop(0, CH // LANES, per_group, 0)

            def pp_update(i, sl):
                gb = (i // LANES) * LANES
                wvec = wown_v[pl.ds(gb, LANES)]
                picks = plsc.load_gather(
                    ering, [jnp.full((LANES,), sl, jnp.int32), wvec])
                cur = pp_v[pl.ds(gb, LANES)]
                pp_v[pl.ds(gb, LANES)] = jnp.where(lane == i % LANES, picks, cur)

            for j in range(NB):
                issue(j, j)
            issue_e(0, 0)
            issue_e(1, 1)

            def body(cc, _):
                c0 = NB * cc
                a = 2 * cc
                for j in range(NB):
                    c = c0 + j
                    i_s = a + j // nch
                    sl = j // nch
                    if j % nch == 0:
                        wait_e(i_s, sl)
                    wait_in(c, j)
                    pl.when(cc > 0)(lambda c=c, j=j: wb_wait(c - NB, j))
                    compute(c, j, sl)
                    wb(c, j)
                    if j % nch == nch - 1:
                        pp_update(i_s, sl)
                        pl.when(i_s + 2 < bpw)(
                            lambda i_s=i_s, sl=sl: issue_e(i_s + 2, sl))
                    pl.when(cc < nchk // NB - 1)(lambda c=c, j=j: issue(c + NB, j))
                return 0

            lax.fori_loop(0, nchk // NB, body, 0)
            for j in range(NB):
                wb_wait(nchk - NB + j, j)

        run_bank(m1_hbm, dt_hbm, ft_v, et_hbm, ppt_v)
        run_bank(m2_hbm, ds_hbm, fs_v, es_hbm, pps_v)

        pltpu.sync_copy(ppt_v, ppt_hbm.at[pl.ds(s0, bpw)])
        pltpu.sync_copy(pps_v, pps_hbm.at[pl.ds(s0, bpw)])

    return k(cidx_flat, idx, wtab, m1, m2, f_t, f_s, e_t, e_s)


def _loss_body(dt_ref, ds_ref, ppt_ref, pps_ref, o_ref, *, n_data, n_tot, b):
    residual = float(n_tot // b - 1) / n_data
    out = 0.0
    for pp_ref, dn_ref in ((pps_ref, ds_ref), (ppt_ref, dt_ref)):
        a0 = jnp.exp(pp_ref[...] / TEMP)
        an = jnp.exp(dn_ref[...] / TEMP)
        z = (jnp.sum(a0) + jnp.sum(an)) / n_tot * n_data
        o0 = a0 / z
        on = an / z
        ld1 = jnp.sum(jnp.log(o0 / (o0 + residual + EPS)))
        ld0 = jnp.sum(jnp.log(residual / (on + residual + EPS)))
        out = out - (ld1 + ld0) / b
    o_ref[...] = jnp.full((1, 1), 1.0) * out


def _loss(dots_t, dots_s, pp_t, pp_s, n_data):
    B = pp_t.shape[0] * pp_t.shape[1]
    n_tot = B + dots_t.shape[0] * dots_t.shape[1]
    body = functools.partial(_loss_body, n_data=n_data, n_tot=n_tot, b=B)
    return pl.pallas_call(
        body,
        out_shape=jax.ShapeDtypeStruct((1, 1), jnp.float32),
    )(dots_t, dots_s, pp_t, pp_s)


def kernel(fs_s, fs_t, idx, contrast_idx, W_s, b_s, W_t, b_t, memory_v1, memory_v2):
    B, K = contrast_idx.shape
    n_data = memory_v1.shape[0]
    idx32 = idx.astype(jnp.int32)
    cidx_flat = contrast_idx.reshape(-1).astype(jnp.int32)
    wtab = jnp.full((n_data,), -1, jnp.int32).at[idx32].set(
        jnp.arange(B, dtype=jnp.int32))

    f_s, f_t = _proj(fs_s, W_s, b_s, fs_t, W_t, b_t)
    pos1, pos2 = _pos_gather(idx32, memory_v1, memory_v2)
    e_t, e_s = _prep(pos1, pos2, f_s, f_t)
    dt_flat, ds_flat, pp_t, pp_s = _sc_main(
        cidx_flat, idx32, wtab, memory_v1, memory_v2, f_t, f_s, e_t, e_s)
    out = _loss(dt_flat.reshape(B * 2, K // 2), ds_flat.reshape(B * 2, K // 2),
                pp_t.reshape(8, B // 8), pp_s.reshape(8, B // 8), n_data)
    return out.reshape(())


# speedup vs baseline: 1.2965x; 1.0137x over previous
import functools

import jax
import jax.numpy as jnp
from jax import lax
from jax.experimental import pallas as pl
from jax.experimental.pallas import tpu as pltpu
from jax.experimental.pallas import tpu_sc as plsc

TEMP = 0.07
MOM = 0.5
EPS = 1e-07

NC = 2
NS = 16
NW = NC * NS
LANES = 16

_GDN = lax.GatherDimensionNumbers(
    offset_dims=(), collapsed_slice_dims=(0,), start_index_map=(0,))


def _shuffle(x, idx):
    return lax.gather(x, idx[:, None], _GDN, slice_sizes=(1,),
                      mode=lax.GatherScatterMode.PROMISE_IN_BOUNDS)


def _projprep_body(xs_ref, ws_ref, bs_ref, xt_ref, wt_ref, bt_ref, p1_ref, p2_ref,
                   os_ref, ot_ref, et_ref, es_ref):
    dn = (((1,), (1,)), ((), ()))
    fs = []
    for x_ref, w_ref, b_ref, o_ref in ((xs_ref, ws_ref, bs_ref, os_ref),
                                       (xt_ref, wt_ref, bt_ref, ot_ref)):
        y = lax.dot_general(x_ref[...], w_ref[...], dn,
                            preferred_element_type=jnp.float32,
                            precision=lax.Precision.HIGHEST)
        y = y + b_ref[...]
        n = jnp.sqrt(jnp.sum(y * y, axis=1, keepdims=True))
        y = y / jnp.maximum(n, 1e-12)
        o_ref[...] = y
        fs.append(y)
    f_s, f_t = fs
    u_s = p1_ref[...] * MOM + f_s * (1.0 - MOM)
    u_s = u_s / jnp.sqrt(jnp.sum(u_s * u_s, axis=1, keepdims=True))
    u_t = p2_ref[...] * MOM + f_t * (1.0 - MOM)
    u_t = u_t / jnp.sqrt(jnp.sum(u_t * u_t, axis=1, keepdims=True))
    et_ref[...] = lax.dot_general(f_t, u_s, dn,
                                  preferred_element_type=jnp.float32,
                                  precision=lax.Precision.HIGHEST)
    es_ref[...] = lax.dot_general(f_s, u_t, dn,
                                  preferred_element_type=jnp.float32,
                                  precision=lax.Precision.HIGHEST)


def _projprep(fs_s, w_s, b_s, fs_t, w_t, b_t, pos1, pos2):
    B = fs_s.shape[0]
    F = w_s.shape[0]
    return pl.pallas_call(
        _projprep_body,
        out_shape=[jax.ShapeDtypeStruct((B, F), jnp.float32),
                   jax.ShapeDtypeStruct((B, F), jnp.float32),
                   jax.ShapeDtypeStruct((B, B), jnp.float32),
                   jax.ShapeDtypeStruct((B, B), jnp.float32)],
    )(fs_s, w_s, b_s.reshape(1, F), fs_t, w_t, b_t.reshape(1, F), pos1, pos2)


def _pos_gather(idx, m1, m2):
    B = idx.shape[0]
    F = m1.shape[1]
    bpw = B // NW
    mesh = plsc.VectorSubcoreMesh(core_axis_name="c", subcore_axis_name="s")

    @functools.partial(
        pl.kernel, mesh=mesh,
        compiler_params=pltpu.CompilerParams(use_tc_tiling_on_sc=False, needs_layout_passes=False),
        out_type=[jax.ShapeDtypeStruct((B, F), jnp.float32),
                  jax.ShapeDtypeStruct((B, F), jnp.float32)],
        scratch_types=[pltpu.VMEM((bpw,), jnp.int32),
                       pltpu.VMEM((bpw, F), jnp.float32),
                       pltpu.SemaphoreType.DMA],
    )
    def k(idx_hbm, m1_hbm, m2_hbm, o1_hbm, o2_hbm, idx_v, rows_v, sem):
        wid = lax.axis_index("s") * NC + lax.axis_index("c")
        base = wid * bpw
        pltpu.sync_copy(idx_hbm.at[pl.ds(base, bpw)], idx_v)
        pltpu.async_copy(m1_hbm.at[idx_v], rows_v, sem).wait()
        pltpu.sync_copy(rows_v, o1_hbm.at[pl.ds(base, bpw)])
        pltpu.async_copy(m2_hbm.at[idx_v], rows_v, sem).wait()
        pltpu.sync_copy(rows_v, o2_hbm.at[pl.ds(base, bpw)])

    return k(idx, m1, m2)


def _sc_main(cidx_flat, idx, wtab, m1, m2, f_t, f_s, e_t, e_s):
    B, K = idx.shape[0], cidx_flat.shape[0] // idx.shape[0]
    F = m1.shape[1]
    bpw = B // NW
    CH = 128
    nch = K // CH
    nchk = bpw * nch
    NB = 4
    mesh = plsc.VectorSubcoreMesh(core_axis_name="c", subcore_axis_name="s")

    @functools.partial(
        pl.kernel, mesh=mesh,
        compiler_params=pltpu.CompilerParams(use_tc_tiling_on_sc=False, needs_layout_passes=False),
        out_type=[jax.ShapeDtypeStruct((B * K,), jnp.float32),
                  jax.ShapeDtypeStruct((B * K,), jnp.float32),
                  jax.ShapeDtypeStruct((B,), jnp.float32),
                  jax.ShapeDtypeStruct((B,), jnp.float32)],
        scratch_types=([pltpu.VMEM((bpw, F), jnp.float32),
                        pltpu.VMEM((bpw, F), jnp.float32),
                        pltpu.VMEM((bpw,), jnp.int32),
                        pltpu.VMEM((bpw,), jnp.int32),
                        pltpu.VMEM((bpw * K,), jnp.int32),
                        pltpu.VMEM((2, B), jnp.float32),
                        pltpu.VMEM((bpw,), jnp.float32),
                        pltpu.VMEM((bpw,), jnp.float32)]
                       + [pltpu.VMEM((CH,), jnp.int32)] * NB
                       + [pltpu.VMEM((CH, F), jnp.float32)] * NB
                       + [pltpu.VMEM((CH,), jnp.float32)] * NB
                       + [pltpu.SemaphoreType.DMA] * (1 + 3 * NB + 2)),
    )
    def k(cidx_hbm, idx_hbm, wtab_hbm, m1_hbm, m2_hbm, ft_hbm, fs_hbm,
          et_hbm, es_hbm, dt_hbm, ds_hbm, ppt_hbm, pps_hbm,
          ft_v, fs_v, own_v, wown_v, ci_all, ering, ppt_v, pps_v,
          wc0, wc1, wc2, wc3, rows0, rows1, rows2, rows3,
          dots0, dots1, dots2, dots3,
          sem, semr0, semr1, semr2, semr3, semw0, semw1, semw2, semw3,
          semd0, semd1, semd2, semd3, seme0, seme1):
        wc_bufs = (wc0, wc1, wc2, wc3)
        rows_bufs = (rows0, rows1, rows2, rows3)
        dots_bufs = (dots0, dots1, dots2, dots3)
        semr_t = (semr0, semr1, semr2, semr3)
        semw_t = (semw0, semw1, semw2, semw3)
        semd_t = (semd0, semd1, semd2, semd3)
        seme_t = (seme0, seme1)

        wid = lax.axis_index("s") * NC + lax.axis_index("c")
        s0 = wid * bpw
        pltpu.sync_copy(ft_hbm.at[pl.ds(s0, bpw)], ft_v)
        pltpu.sync_copy(fs_hbm.at[pl.ds(s0, bpw)], fs_v)
        pltpu.sync_copy(idx_hbm.at[pl.ds(s0, bpw)], own_v)
        pltpu.sync_copy(cidx_hbm.at[pl.ds(s0 * K, bpw * K)], ci_all)
        pltpu.async_copy(wtab_hbm.at[own_v], wown_v, sem).wait()

        lane = jnp.arange(LANES, dtype=jnp.int32)

        def run_bank(mem_hbm, o_hbm, f_v, e_hbm, pp_v):
            def locs(c):
                i = c // nch
                lbase = i * K + (c - i * nch) * CH
                return i, lbase

            def issue(c, j):
                _, lbase = locs(c)
                idxref = ci_all.at[pl.ds(lbase, CH)]
                pltpu.async_copy(wtab_hbm.at[idxref], wc_bufs[j], semw_t[j])
                pltpu.async_copy(mem_hbm.at[idxref], rows_bufs[j], semr_t[j])

            def wait_in(c, j):
                _, lbase = locs(c)
                idxref = ci_all.at[pl.ds(lbase, CH)]
                pltpu.make_async_copy(wtab_hbm.at[idxref], wc_bufs[j], semw_t[j]).wait()
                pltpu.make_async_copy(mem_hbm.at[idxref], rows_bufs[j], semr_t[j]).wait()

            def wb(c, j):
                _, lbase = locs(c)
                pltpu.async_copy(dots_bufs[j],
                                 o_hbm.at[pl.ds(s0 * K + lbase, CH)], semd_t[j])

            def wb_wait(c, j):
                _, lbase = locs(c)
                pltpu.make_async_copy(
                    dots_bufs[j], o_hbm.at[pl.ds(s0 * K + lbase, CH)],
                    semd_t[j]).wait()

            def issue_e(i, sl):
                pltpu.async_copy(e_hbm.at[s0 + i], ering.at[sl], seme_t[sl])

            def wait_e(i, sl):
                pltpu.make_async_copy(
                    e_hbm.at[s0 + i], ering.at[sl], seme_t[sl]).wait()

            def compute(c, j, sl):
                i, _ = locs(c)
                rows_b, wc_b, dots_b = rows_bufs[j], wc_bufs[j], dots_bufs[j]
                fsegs = [f_v[i, pl.ds(ss * LANES, LANES)]
                         for ss in range(F // LANES)]
                esplat = jnp.full((LANES,), sl, jnp.int32)

                def per_group(rg, _):
                    dots = jnp.zeros((LANES,), jnp.float32)
                    for rr in range(LANES):
                        r = rg * LANES + rr
                        acc = rows_b[r, pl.ds(0, LANES)] * fsegs[0]
                        for ss in range(1, F // LANES):
                            acc = acc + (rows_b[r, pl.ds(ss * LANES, LANES)]
                                         * fsegs[ss])
                        for step in (1, 2, 4, 8):
                            acc = acc + _shuffle(acc, lane ^ step)
                        dots = jnp.where(lane == rr, acc, dots)
                    wvec = wc_b[pl.ds(rg * LANES, LANES)]
                    hit = wvec >= 0
                    corr = plsc.load_gather(ering, [esplat, jnp.maximum(wvec, 0)])
                    dots_b[pl.ds(rg * LANES, LANES)] = jnp.where(hit, corr, dots)
                    return 0

                lax.fori_loop(0, CH // LANES, per_group, 0)

            def pp_update(i, sl):
                gb = (i // LANES) * LANES
                wvec = wown_v[pl.ds(gb, LANES)]
                picks = plsc.load_gather(
                    ering, [jnp.full((LANES,), sl, jnp.int32), wvec])
                cur = pp_v[pl.ds(gb, LANES)]
                pp_v[pl.ds(gb, LANES)] = jnp.where(lane == i % LANES, picks, cur)

            for j in range(NB):
                issue(j, j)
            issue_e(0, 0)
            issue_e(1, 1)

            def body(cc, _):
                c0 = NB * cc
                a = 2 * cc
                for j in range(NB):
                    c = c0 + j
                    i_s = a + j // nch
                    sl = j // nch
                    if j % nch == 0:
                        wait_e(i_s, sl)
                    wait_in(c, j)
                    pl.when(cc > 0)(lambda c=c, j=j: wb_wait(c - NB, j))
                    compute(c, j, sl)
                    wb(c, j)
                    if j % nch == nch - 1:
                        pp_update(i_s, sl)
                        pl.when(i_s + 2 < bpw)(
                            lambda i_s=i_s, sl=sl: issue_e(i_s + 2, sl))
                    pl.when(cc < nchk // NB - 1)(lambda c=c, j=j: issue(c + NB, j))
                return 0

            lax.fori_loop(0, nchk // NB, body, 0)
            for j in range(NB):
                wb_wait(nchk - NB + j, j)

        run_bank(m1_hbm, dt_hbm, ft_v, et_hbm, ppt_v)
        run_bank(m2_hbm, ds_hbm, fs_v, es_hbm, pps_v)

        pltpu.sync_copy(ppt_v, ppt_hbm.at[pl.ds(s0, bpw)])
        pltpu.sync_copy(pps_v, pps_hbm.at[pl.ds(s0, bpw)])

    return k(cidx_flat, idx, wtab, m1, m2, f_t, f_s, e_t, e_s)


def _loss_body(dt_ref, ds_ref, ppt_ref, pps_ref, o_ref, *, n_data, n_tot, b):
    residual = float(n_tot // b - 1) / n_data
    out = 0.0
    for pp_ref, dn_ref in ((pps_ref, ds_ref), (ppt_ref, dt_ref)):
        a0 = jnp.exp(pp_ref[...] / TEMP)
        an = jnp.exp(dn_ref[...] / TEMP)
        z = (jnp.sum(a0) + jnp.sum(an)) / n_tot * n_data
        o0 = a0 / z
        on = an / z
        ld1 = jnp.sum(jnp.log(o0 / (o0 + residual + EPS)))
        ld0 = jnp.sum(jnp.log(residual / (on + residual + EPS)))
        out = out - (ld1 + ld0) / b
    o_ref[...] = jnp.full((1, 1), 1.0) * out


def _loss(dots_t, dots_s, pp_t, pp_s, n_data):
    B = pp_t.shape[0] * pp_t.shape[1]
    n_tot = B + dots_t.shape[0] * dots_t.shape[1]
    body = functools.partial(_loss_body, n_data=n_data, n_tot=n_tot, b=B)
    return pl.pallas_call(
        body,
        out_shape=jax.ShapeDtypeStruct((1, 1), jnp.float32),
    )(dots_t, dots_s, pp_t, pp_s)


def kernel(fs_s, fs_t, idx, contrast_idx, W_s, b_s, W_t, b_t, memory_v1, memory_v2):
    B, K = contrast_idx.shape
    n_data = memory_v1.shape[0]
    idx32 = idx.astype(jnp.int32)
    cidx_flat = contrast_idx.reshape(-1).astype(jnp.int32)
    wtab = jnp.full((n_data,), -1, jnp.int32).at[idx32].set(
        jnp.arange(B, dtype=jnp.int32))

    pos1, pos2 = _pos_gather(idx32, memory_v1, memory_v2)
    f_s, f_t, e_t, e_s = _projprep(fs_s, W_s, b_s, fs_t, W_t, b_t, pos1, pos2)
    dt_flat, ds_flat, pp_t, pp_s = _sc_main(
        cidx_flat, idx32, wtab, memory_v1, memory_v2, f_t, f_s, e_t, e_s)
    out = _loss(dt_flat.reshape(B * 2, K // 2), ds_flat.reshape(B * 2, K // 2),
                pp_t.reshape(8, B // 8), pp_s.reshape(8, B // 8), n_data)
    return out.reshape(())
